# baseline (speedup 1.0000x reference)
import functools

import jax
import jax.numpy as jnp
from jax import lax
from jax.experimental import pallas as pl
from jax.experimental.pallas import tpu as pltpu
from jax.experimental.pallas import tpu_sc as plsc

N = 10000
NPAD = 10240
K = 8
EPAD = NPAD * K

_RLIN = 400
_RKNN = 1000
_CKNN = 1280
_RC = 200
_PADVAL = 1e17

_BIGF = 3e38
_BIGI = 2 ** 30

_MM = jnp.bfloat16


def _lin1_body(x_ref, w1_ref, b1_ref, w2_ref, b2_ref, h_ref, sq_ref):
    t = jnp.dot(x_ref[...].astype(_MM), w1_ref[...],
                preferred_element_type=jnp.float32)
    t = jnp.maximum(t + b1_ref[...], 0.0)
    h = jnp.dot(t.astype(_MM), w2_ref[...],
                preferred_element_type=jnp.float32) + b2_ref[...]
    h_ref[...] = h
    sq_ref[...] = jnp.sum(h * h, axis=1, keepdims=True)


def _linear1(x, W1, b1, W2, b2):
    r = _RLIN
    return pl.pallas_call(
        _lin1_body,
        grid=(N // r,),
        in_specs=[
            pl.BlockSpec((r, 128), lambda i: (i, 0)),
            pl.BlockSpec((128, 1024), lambda i: (0, 0)),
            pl.BlockSpec((1, 1024), lambda i: (0, 0)),
            pl.BlockSpec((1024, 256), lambda i: (0, 0)),
            pl.BlockSpec((1, 256), lambda i: (0, 0)),
        ],
        out_specs=[pl.BlockSpec((r, 256), lambda i: (i, 0)),
                   pl.BlockSpec((r, 1), lambda i: (i, 0))],
        out_shape=[jax.ShapeDtypeStruct((N, 256), jnp.float32),
                   jax.ShapeDtypeStruct((N, 1), jnp.float32)],
    )(x, W1.astype(_MM), b1[None, :], W2.astype(_MM), b2[None, :])


def _uv_body(h_ref, wa_ref, ba_ref, wb_ref, u_ref, v_ref):
    hb = h_ref[...].astype(_MM)
    u_ref[...] = jnp.dot(hb, wa_ref[...],
                         preferred_element_type=jnp.float32) + ba_ref[...]
    v_ref[...] = jnp.dot(hb, wb_ref[...], preferred_element_type=jnp.float32)


def _uv(h, Wa, ba, Wb):
    din = h.shape[1]
    hdim = Wa.shape[1]
    r = _RLIN
    return pl.pallas_call(
        _uv_body,
        grid=(N // r,),
        in_specs=[
            pl.BlockSpec((r, din), lambda i: (i, 0)),
            pl.BlockSpec((din, hdim), lambda i: (0, 0)),
            pl.BlockSpec((1, hdim), lambda i: (0, 0)),
            pl.BlockSpec((din, hdim), lambda i: (0, 0)),
        ],
        out_specs=[
            pl.BlockSpec((r, hdim), lambda i: (i, 0)),
            pl.BlockSpec((r, hdim), lambda i: (i, 0)),
        ],
        out_shape=[
            jax.ShapeDtypeStruct((N, hdim), jnp.float32),
            jax.ShapeDtypeStruct((N, hdim), jnp.float32),
        ],
    )(h, Wa.astype(_MM), ba[None, :], Wb.astype(_MM))


def _knn_body(hr_ref, hct_ref, sqr_ref, sqc_ref, idx_ref, rv_ref, ri_ref,
              pm_ref, px_ref, im_ref, ix_ref, tv_ref, ti_ref, cv_ref, ci_ref):
    j = pl.program_id(1)
    ncol = NPAD // _CKNN
    c2 = _CKNN // 2

    @pl.when(j == 0)
    def _():
        rv_ref[...] = jnp.full((_RKNN, K), _BIGF, jnp.float32)
        ri_ref[...] = jnp.full((_RKNN, K), _BIGI, jnp.int32)

    dots = jnp.dot(hr_ref[...], hct_ref[...],
                   preferred_element_type=jnp.float32)
    tile = (sqr_ref[...] + sqc_ref[...]) - 2.0 * dots

    piota = lax.broadcasted_iota(jnp.int32, (_RKNN, c2), 1)
    k8 = lax.broadcasted_iota(jnp.int32, (_RKNN, K), 1)
    lo = tile[:, :c2]
    hi = tile[:, c2:]
    takel = lo <= hi
    base = j * _CKNN
    pm_ref[...] = jnp.where(takel, lo, hi)
    px_ref[...] = jnp.where(takel, hi, lo)
    im_ref[...] = jnp.where(takel, piota + base, piota + (base + c2))
    ix_ref[...] = jnp.where(takel, piota + (base + c2), piota + base)

    def tile_round(t, carry):
        pv = pm_ref[...]
        m = jnp.min(pv, axis=1, keepdims=True)
        sel = jnp.where(pv == m, piota, _BIGI)
        ap = jnp.min(sel, axis=1, keepdims=True)
        hit = piota == ap
        widx = jnp.min(jnp.where(hit, im_ref[...], _BIGI),
                       axis=1, keepdims=True)
        tv_ref[...] = jnp.where(k8 == t, m, tv_ref[...])
        ti_ref[...] = jnp.where(k8 == t, widx, ti_ref[...])
        pm_ref[...] = jnp.where(hit, px_ref[...], pv)
        im_ref[...] = jnp.where(hit, ix_ref[...], im_ref[...])
        px_ref[...] = jnp.where(hit, _BIGF, px_ref[...])
        return carry

    lax.fori_loop(0, K, tile_round, 0)

    cv_ref[...] = jnp.concatenate([rv_ref[...], tv_ref[...]], axis=1)
    ci_ref[...] = jnp.concatenate([ri_ref[...], ti_ref[...]], axis=1)
    miota = lax.broadcasted_iota(jnp.int32, (_RKNN, 2 * K), 1)

    def merge_round(t, carry):
        cv = cv_ref[...]
        m = jnp.min(cv, axis=1, keepdims=True)
        sel = jnp.where(cv == m, miota, _BIGI)
        am = jnp.min(sel, axis=1, keepdims=True)
        pick = jnp.min(jnp.where(miota == am, ci_ref[...], _BIGI),
                       axis=1, keepdims=True)
        rv_ref[...] = jnp.where(k8 == t, m, rv_ref[...])
        ri_ref[...] = jnp.where(k8 == t, pick, ri_ref[...])
        cv_ref[...] = jnp.where(miota == am, _BIGF, cv)
        return carry

    lax.fori_loop(0, K, merge_round, 0)

    @pl.when(j == ncol - 1)
    def _():
        idx_ref[...] = ri_ref[...]


def _knn(hpb, sq):
    d = hpb.shape[1]
    hptb = hpb.T
    sqt = jnp.pad(sq, ((0, NPAD - N), (0, 0)), constant_values=1e36).T
    return pl.pallas_call(
        _knn_body,
        grid=(N // _RKNN, NPAD // _CKNN),
        in_specs=[
            pl.BlockSpec((_RKNN, d), lambda i, j: (i, 0)),
            pl.BlockSpec((d, _CKNN), lambda i, j: (0, j)),
            pl.BlockSpec((_RKNN, 1), lambda i, j: (i, 0)),
            pl.BlockSpec((1, _CKNN), lambda i, j: (0, j)),
        ],
        out_specs=pl.BlockSpec((_RKNN, K), lambda i, j: (i, 0)),
        out_shape=jax.ShapeDtypeStruct((N, K), jnp.int32),
        scratch_shapes=[pltpu.VMEM((_RKNN, K), jnp.float32),
                        pltpu.VMEM((_RKNN, K), jnp.int32),
                        pltpu.VMEM((_RKNN, _CKNN // 2), jnp.float32),
                        pltpu.VMEM((_RKNN, _CKNN // 2), jnp.float32),
                        pltpu.VMEM((_RKNN, _CKNN // 2), jnp.int32),
                        pltpu.VMEM((_RKNN, _CKNN // 2), jnp.int32),
                        pltpu.VMEM((_RKNN, K), jnp.float32),
                        pltpu.VMEM((_RKNN, K), jnp.int32),
                        pltpu.VMEM((_RKNN, 2 * K), jnp.float32),
                        pltpu.VMEM((_RKNN, 2 * K), jnp.int32)],
        compiler_params=pltpu.CompilerParams(
            dimension_semantics=("parallel", "arbitrary")),
    )(hpb, hptb, sq, sqt)


def _sc_gather_body(chunk, per_w, nchunk, table, idxf, out, idx_v, rows_v, sem):
    c = lax.axis_index("c")
    s = lax.axis_index("s")
    wid = s * 2 + c
    base = wid * per_w

    def step(jj, carry):
        off = base + jj * chunk
        pltpu.sync_copy(idxf.at[pl.ds(off, chunk)], idx_v)
        pltpu.async_copy(table.at[idx_v], rows_v, sem).wait()
        pltpu.sync_copy(rows_v, out.at[pl.ds(off, chunk)])
        return carry

    lax.fori_loop(0, nchunk, step, 0)


def _sc_gather(V, idx_flat):
    d = V.shape[1]
    per_w = EPAD // 32
    chunk = 128 if d <= 512 else 32
    nchunk = per_w // chunk
    mesh = plsc.VectorSubcoreMesh(core_axis_name="c", subcore_axis_name="s")
    f = pl.kernel(
        functools.partial(_sc_gather_body, chunk, per_w, nchunk),
        mesh=mesh,
        out_type=jax.ShapeDtypeStruct((EPAD, d), jnp.float32),
        scratch_types=[
            pltpu.VMEM((chunk,), jnp.int32),
            pltpu.VMEM((chunk, d), jnp.float32),
            pltpu.SemaphoreType.DMA,
        ],
    )
    return f(V, idx_flat)


def _conv1_body(xi_ref, g_ref, w3t_ref, w3b_ref, b3_ref, w4_ref, b4_ref,
                out_ref, sq_ref, acc_ref):
    kk = pl.program_id(1)
    xi = xi_ref[...]
    db = g_ref[0] - xi
    hid = jnp.dot(xi.astype(_MM), w3t_ref[...],
                  preferred_element_type=jnp.float32)
    hid = hid + jnp.dot(db.astype(_MM), w3b_ref[...],
                        preferred_element_type=jnp.float32)
    hid = jnp.maximum(hid + b3_ref[...], 0.0)
    part = jnp.dot(hid.astype(_MM), w4_ref[...],
                   preferred_element_type=jnp.float32)

    @pl.when(kk == 0)
    def _():
        acc_ref[...] = part

    @pl.when(kk > 0)
    def _():
        acc_ref[...] = jnp.maximum(acc_ref[...], part)

    @pl.when(kk == K - 1)
    def _():
        out = acc_ref[...] + b4_ref[...]
        out_ref[...] = out
        sq_ref[...] = jnp.sum(out * out, axis=1, keepdims=True)


def _conv1(h1, g1, W3, b3, W4, b4):
    din = h1.shape[1]
    hdim = W3.shape[1]
    dout = W4.shape[1]
    r = _RC
    return pl.pallas_call(
        _conv1_body,
        grid=(N // r, K),
        in_specs=[
            pl.BlockSpec((r, din), lambda i, kk: (i, 0)),
            pl.BlockSpec((1, r, din), lambda i, kk: (kk, i, 0)),
            pl.BlockSpec((din, hdim), lambda i, kk: (0, 0)),
            pl.BlockSpec((din, hdim), lambda i, kk: (0, 0)),
            pl.BlockSpec((1, hdim), lambda i, kk: (0, 0)),
            pl.BlockSpec((hdim, dout), lambda i, kk: (0, 0)),
            pl.BlockSpec((1, dout), lambda i, kk: (0, 0)),
        ],
        out_specs=[pl.BlockSpec((r, dout), lambda i, kk: (i, 0)),
                   pl.BlockSpec((r, 1), lambda i, kk: (i, 0))],
        out_shape=[jax.ShapeDtypeStruct((N, dout), jnp.float32),
                   jax.ShapeDtypeStruct((N, 1), jnp.float32)],
        scratch_shapes=[pltpu.VMEM((r, dout), jnp.float32)],
        compiler_params=pltpu.CompilerParams(
            dimension_semantics=("parallel", "arbitrary")),
    )(h1, g1, W3[:din].astype(_MM), W3[din:].astype(_MM), b3[None, :],
      W4.astype(_MM), b4[None, :])


def _conv2_body(u_ref, g_ref, w6_ref, b6_ref, out_ref, acc_ref):
    kk = pl.program_id(1)
    hid = jnp.maximum(u_ref[...] + g_ref[0], 0.0)
    part = jnp.dot(hid.astype(_MM), w6_ref[...],
                   preferred_element_type=jnp.float32)

    @pl.when(kk == 0)
    def _():
        acc_ref[...] = part

    @pl.when(kk > 0)
    def _():
        acc_ref[...] = jnp.maximum(acc_ref[...], part)

    @pl.when(kk == K - 1)
    def _():
        out_ref[...] = acc_ref[...] + b6_ref[...]


def _conv2(u, g, W6, b6):
    hdim = u.shape[1]
    dout = W6.shape[1]
    r = _RC
    return pl.pallas_call(
        _conv2_body,
        grid=(N // r, K),
        in_specs=[
            pl.BlockSpec((r, hdim), lambda i, kk: (i, 0)),
            pl.BlockSpec((1, r, hdim), lambda i, kk: (kk, i, 0)),
            pl.BlockSpec((hdim, dout), lambda i, kk: (0, 0)),
            pl.BlockSpec((1, dout), lambda i, kk: (0, 0)),
        ],
        out_specs=pl.BlockSpec((r, dout), lambda i, kk: (i, 0)),
        out_shape=jax.ShapeDtypeStruct((N, dout), jnp.float32),
        scratch_shapes=[pltpu.VMEM((r, dout), jnp.float32)],
        compiler_params=pltpu.CompilerParams(
            dimension_semantics=("parallel", "arbitrary")),
    )(u, g, W6.astype(_MM), b6[None, :])


def kernel(x, W1, b1, W2, b2, W3, b3, W4, b4, W5, b5, W6, b6):
    d2 = W5.shape[0] // 2
    W5a = W5[:d2] - W5[d2:]
    W5b = W5[d2:]

    h1, sq1 = _linear1(x, W1, b1, W2, b2)
    h1pb = jnp.pad(h1.astype(_MM), ((0, NPAD - N), (0, 0)),
                   constant_values=_PADVAL)
    idx1 = _knn(h1pb, sq1)
    idxf1 = jnp.pad(idx1, ((0, NPAD - N), (0, 0))).T.reshape(-1)
    g1 = _sc_gather(h1, idxf1).reshape(K, NPAD, h1.shape[1])
    h2, sq2 = _conv1(h1, g1, W3, b3, W4, b4)

    u2, v2 = _uv(h2, W5a, b5, W5b)
    h2pb = jnp.pad(h2.astype(_MM), ((0, NPAD - N), (0, 0)),
                   constant_values=_PADVAL)
    idx2 = _knn(h2pb, sq2)
    idxf2 = jnp.pad(idx2, ((0, NPAD - N), (0, 0))).T.reshape(-1)
    g2 = _sc_gather(v2, idxf2).reshape(K, NPAD, v2.shape[1])
    return _conv2(u2, g2, W6, b6)


# speedup vs baseline: 1.0274x; 1.0274x over previous
import functools

import jax
import jax.numpy as jnp
from jax import lax
from jax.experimental import pallas as pl
from jax.experimental.pallas import tpu as pltpu
from jax.experimental.pallas import tpu_sc as plsc

N = 10000
NPAD = 10240
K = 8
EPAD = NPAD * K

_RLIN = 400
_RKNN = 1000
_CKNN = 1280
_RC = 200
_PADVAL = 1e17

_BIGF = 3e38
_BIGI = 2 ** 30

_MM = jnp.bfloat16


def _lin1_body(x_ref, w1_ref, b1_ref, w2_ref, b2_ref, h_ref, sq_ref):
    t = jnp.dot(x_ref[...].astype(_MM), w1_ref[...],
                preferred_element_type=jnp.float32)
    t = jnp.maximum(t + b1_ref[...], 0.0)
    h = jnp.dot(t.astype(_MM), w2_ref[...],
                preferred_element_type=jnp.float32) + b2_ref[...]
    h_ref[...] = h
    sq_ref[...] = jnp.sum(h * h, axis=1, keepdims=True)


def _linear1(x, W1, b1, W2, b2):
    r = _RLIN
    return pl.pallas_call(
        _lin1_body,
        grid=(N // r,),
        in_specs=[
            pl.BlockSpec((r, 128), lambda i: (i, 0)),
            pl.BlockSpec((128, 1024), lambda i: (0, 0)),
            pl.BlockSpec((1, 1024), lambda i: (0, 0)),
            pl.BlockSpec((1024, 256), lambda i: (0, 0)),
            pl.BlockSpec((1, 256), lambda i: (0, 0)),
        ],
        out_specs=[pl.BlockSpec((r, 256), lambda i: (i, 0)),
                   pl.BlockSpec((r, 1), lambda i: (i, 0))],
        out_shape=[jax.ShapeDtypeStruct((N, 256), jnp.float32),
                   jax.ShapeDtypeStruct((N, 1), jnp.float32)],
    )(x, W1.astype(_MM), b1[None, :], W2.astype(_MM), b2[None, :])


def _uv_body(h_ref, wa_ref, ba_ref, wb_ref, u_ref, v_ref):
    hb = h_ref[...].astype(_MM)
    u_ref[...] = jnp.dot(hb, wa_ref[...],
                         preferred_element_type=jnp.float32) + ba_ref[...]
    v_ref[...] = jnp.dot(hb, wb_ref[...], preferred_element_type=jnp.float32)


def _uv(h, Wa, ba, Wb):
    din = h.shape[1]
    hdim = Wa.shape[1]
    r = _RLIN
    return pl.pallas_call(
        _uv_body,
        grid=(N // r,),
        in_specs=[
            pl.BlockSpec((r, din), lambda i: (i, 0)),
            pl.BlockSpec((din, hdim), lambda i: (0, 0)),
            pl.BlockSpec((1, hdim), lambda i: (0, 0)),
            pl.BlockSpec((din, hdim), lambda i: (0, 0)),
        ],
        out_specs=[
            pl.BlockSpec((r, hdim), lambda i: (i, 0)),
            pl.BlockSpec((r, hdim), lambda i: (i, 0)),
        ],
        out_shape=[
            jax.ShapeDtypeStruct((N, hdim), jnp.float32),
            jax.ShapeDtypeStruct((N, hdim), jnp.float32),
        ],
    )(h, Wa.astype(_MM), ba[None, :], Wb.astype(_MM))


def _knn_body(hr_ref, hct_ref, sqr_ref, sqc_ref, idx_ref, rv_ref, ri_ref,
              tile_ref, tv_ref, ti_ref, cv_ref, ci_ref):
    j = pl.program_id(1)
    ncol = NPAD // _CKNN

    @pl.when(j == 0)
    def _():
        rv_ref[...] = jnp.full((_RKNN, K), _BIGF, jnp.float32)
        ri_ref[...] = jnp.full((_RKNN, K), _BIGI, jnp.int32)

    dots = jnp.dot(hr_ref[...], hct_ref[...],
                   preferred_element_type=jnp.float32)
    tile_ref[...] = (sqr_ref[...] + sqc_ref[...]) - 2.0 * dots

    tiota = lax.broadcasted_iota(jnp.int32, (_RKNN, _CKNN), 1)
    k8 = lax.broadcasted_iota(jnp.int32, (_RKNN, K), 1)

    def tile_round(t, carry):
        tl = tile_ref[...]
        m = jnp.min(tl, axis=1, keepdims=True)
        sel = jnp.where(tl == m, tiota, _BIGI)
        am = jnp.min(sel, axis=1, keepdims=True)
        tv_ref[...] = jnp.where(k8 == t, m, tv_ref[...])
        ti_ref[...] = jnp.where(k8 == t, am + j * _CKNN, ti_ref[...])
        tile_ref[...] = jnp.where(tiota == am, _BIGF, tl)
        return carry

    lax.fori_loop(0, K, tile_round, 0)

    cv_ref[...] = jnp.concatenate([rv_ref[...], tv_ref[...]], axis=1)
    ci_ref[...] = jnp.concatenate([ri_ref[...], ti_ref[...]], axis=1)
    miota = lax.broadcasted_iota(jnp.int32, (_RKNN, 2 * K), 1)

    def merge_round(t, carry):
        cv = cv_ref[...]
        m = jnp.min(cv, axis=1, keepdims=True)
        sel = jnp.where(cv == m, miota, _BIGI)
        am = jnp.min(sel, axis=1, keepdims=True)
        pick = jnp.min(jnp.where(miota == am, ci_ref[...], _BIGI),
                       axis=1, keepdims=True)
        rv_ref[...] = jnp.where(k8 == t, m, rv_ref[...])
        ri_ref[...] = jnp.where(k8 == t, pick, ri_ref[...])
        cv_ref[...] = jnp.where(miota == am, _BIGF, cv)
        return carry

    lax.fori_loop(0, K, merge_round, 0)

    @pl.when(j == ncol - 1)
    def _():
        idx_ref[...] = ri_ref[...]


def _knn(hpb, sq):
    d = hpb.shape[1]
    hptb = hpb.T
    sqt = jnp.pad(sq, ((0, NPAD - N), (0, 0)), constant_values=1e36).T
    return pl.pallas_call(
        _knn_body,
        grid=(N // _RKNN, NPAD // _CKNN),
        in_specs=[
            pl.BlockSpec((_RKNN, d), lambda i, j: (i, 0)),
            pl.BlockSpec((d, _CKNN), lambda i, j: (0, j)),
            pl.BlockSpec((_RKNN, 1), lambda i, j: (i, 0)),
            pl.BlockSpec((1, _CKNN), lambda i, j: (0, j)),
        ],
        out_specs=pl.BlockSpec((_RKNN, K), lambda i, j: (i, 0)),
        out_shape=jax.ShapeDtypeStruct((N, K), jnp.int32),
        scratch_shapes=[pltpu.VMEM((_RKNN, K), jnp.float32),
                        pltpu.VMEM((_RKNN, K), jnp.int32),
                        pltpu.VMEM((_RKNN, _CKNN), jnp.float32),
                        pltpu.VMEM((_RKNN, K), jnp.float32),
                        pltpu.VMEM((_RKNN, K), jnp.int32),
                        pltpu.VMEM((_RKNN, 2 * K), jnp.float32),
                        pltpu.VMEM((_RKNN, 2 * K), jnp.int32)],
        compiler_params=pltpu.CompilerParams(
            dimension_semantics=("parallel", "arbitrary")),
    )(hpb, hptb, sq, sqt)


def _sc_gather_body(chunk, per_w, nchunk, table, idxf, out, idx_v, rows_v, sem):
    c = lax.axis_index("c")
    s = lax.axis_index("s")
    wid = s * 2 + c
    base = wid * per_w

    def step(jj, carry):
        off = base + jj * chunk
        pltpu.sync_copy(idxf.at[pl.ds(off, chunk)], idx_v)
        pltpu.async_copy(table.at[idx_v], rows_v, sem).wait()
        pltpu.sync_copy(rows_v, out.at[pl.ds(off, chunk)])
        return carry

    lax.fori_loop(0, nchunk, step, 0)


def _sc_gather(V, idx_flat):
    d = V.shape[1]
    per_w = EPAD // 32
    chunk = 128 if d <= 512 else 32
    nchunk = per_w // chunk
    mesh = plsc.VectorSubcoreMesh(core_axis_name="c", subcore_axis_name="s")
    f = pl.kernel(
        functools.partial(_sc_gather_body, chunk, per_w, nchunk),
        mesh=mesh,
        out_type=jax.ShapeDtypeStruct((EPAD, d), jnp.float32),
        scratch_types=[
            pltpu.VMEM((chunk,), jnp.int32),
            pltpu.VMEM((chunk, d), jnp.float32),
            pltpu.SemaphoreType.DMA,
        ],
    )
    return f(V, idx_flat)


def _conv1_body(xi_ref, g_ref, w3t_ref, w3b_ref, b3_ref, w4_ref, b4_ref,
                out_ref, sq_ref, acc_ref):
    kk = pl.program_id(1)
    xi = xi_ref[...]
    db = g_ref[0] - xi
    hid = jnp.dot(xi.astype(_MM), w3t_ref[...],
                  preferred_element_type=jnp.float32)
    hid = hid + jnp.dot(db.astype(_MM), w3b_ref[...],
                        preferred_element_type=jnp.float32)
    hid = jnp.maximum(hid + b3_ref[...], 0.0)
    part = jnp.dot(hid.astype(_MM), w4_ref[...],
                   preferred_element_type=jnp.float32)

    @pl.when(kk == 0)
    def _():
        acc_ref[...] = part

    @pl.when(kk > 0)
    def _():
        acc_ref[...] = jnp.maximum(acc_ref[...], part)

    @pl.when(kk == K - 1)
    def _():
        out = acc_ref[...] + b4_ref[...]
        out_ref[...] = out
        sq_ref[...] = jnp.sum(out * out, axis=1, keepdims=True)


def _conv1(h1, g1, W3, b3, W4, b4):
    din = h1.shape[1]
    hdim = W3.shape[1]
    dout = W4.shape[1]
    r = _RC
    return pl.pallas_call(
        _conv1_body,
        grid=(N // r, K),
        in_specs=[
            pl.BlockSpec((r, din), lambda i, kk: (i, 0)),
            pl.BlockSpec((1, r, din), lambda i, kk: (kk, i, 0)),
            pl.BlockSpec((din, hdim), lambda i, kk: (0, 0)),
            pl.BlockSpec((din, hdim), lambda i, kk: (0, 0)),
            pl.BlockSpec((1, hdim), lambda i, kk: (0, 0)),
            pl.BlockSpec((hdim, dout), lambda i, kk: (0, 0)),
            pl.BlockSpec((1, dout), lambda i, kk: (0, 0)),
        ],
        out_specs=[pl.BlockSpec((r, dout), lambda i, kk: (i, 0)),
                   pl.BlockSpec((r, 1), lambda i, kk: (i, 0))],
        out_shape=[jax.ShapeDtypeStruct((N, dout), jnp.float32),
                   jax.ShapeDtypeStruct((N, 1), jnp.float32)],
        scratch_shapes=[pltpu.VMEM((r, dout), jnp.float32)],
        compiler_params=pltpu.CompilerParams(
            dimension_semantics=("parallel", "arbitrary")),
    )(h1, g1, W3[:din].astype(_MM), W3[din:].astype(_MM), b3[None, :],
      W4.astype(_MM), b4[None, :])


def _conv2_body(u_ref, g_ref, w6_ref, b6_ref, out_ref, acc_ref):
    kk = pl.program_id(1)
    hid = jnp.maximum(u_ref[...] + g_ref[0], 0.0)
    part = jnp.dot(hid.astype(_MM), w6_ref[...],
                   preferred_element_type=jnp.float32)

    @pl.when(kk == 0)
    def _():
        acc_ref[...] = part

    @pl.when(kk > 0)
    def _():
        acc_ref[...] = jnp.maximum(acc_ref[...], part)

    @pl.when(kk == K - 1)
    def _():
        out_ref[...] = acc_ref[...] + b6_ref[...]


def _conv2(u, g, W6, b6):
    hdim = u.shape[1]
    dout = W6.shape[1]
    r = _RC
    return pl.pallas_call(
        _conv2_body,
        grid=(N // r, K),
        in_specs=[
            pl.BlockSpec((r, hdim), lambda i, kk: (i, 0)),
            pl.BlockSpec((1, r, hdim), lambda i, kk: (kk, i, 0)),
            pl.BlockSpec((hdim, dout), lambda i, kk: (0, 0)),
            pl.BlockSpec((1, dout), lambda i, kk: (0, 0)),
        ],
        out_specs=pl.BlockSpec((r, dout), lambda i, kk: (i, 0)),
        out_shape=jax.ShapeDtypeStruct((N, dout), jnp.float32),
        scratch_shapes=[pltpu.VMEM((r, dout), jnp.float32)],
        compiler_params=pltpu.CompilerParams(
            dimension_semantics=("parallel", "arbitrary")),
    )(u, g, W6.astype(_MM), b6[None, :])


def kernel(x, W1, b1, W2, b2, W3, b3, W4, b4, W5, b5, W6, b6):
    d2 = W5.shape[0] // 2
    W5a = W5[:d2] - W5[d2:]
    W5b = W5[d2:]

    h1, sq1 = _linear1(x, W1, b1, W2, b2)
    h1pb = jnp.pad(h1.astype(_MM), ((0, NPAD - N), (0, 0)),
                   constant_values=_PADVAL)
    idx1 = _knn(h1pb, sq1)
    idxf1 = jnp.pad(idx1, ((0, NPAD - N), (0, 0))).T.reshape(-1)
    g1 = _sc_gather(h1, idxf1).reshape(K, NPAD, h1.shape[1])
    h2, sq2 = _conv1(h1, g1, W3, b3, W4, b4)

    u2, v2 = _uv(h2, W5a, b5, W5b)
    h2pb = jnp.pad(h2.astype(_MM), ((0, NPAD - N), (0, 0)),
                   constant_values=_PADVAL)
    idx2 = _knn(h2pb, sq2)
    idxf2 = jnp.pad(idx2, ((0, NPAD - N), (0, 0))).T.reshape(-1)
    g2 = _sc_gather(v2, idxf2).reshape(K, NPAD, v2.shape[1])
    return _conv2(u2, g2, W6, b6)
